# baseline (speedup 1.0000x reference)
import functools

import jax
import jax.numpy as jnp
from jax.experimental import pallas as pl
from jax.experimental.pallas import tpu as pltpu
from jax.experimental.pallas import tpu_sc as plsc

_K = 8192
_D = 256
_N = 16384
_MT = 128


def _dist_argmin_body(z_ref, et_ref, sz_ref, se_ref, idx_ref, dsum_ref):
    m = jax.lax.dot_general(
        z_ref[...].astype(jnp.bfloat16), et_ref[...].astype(jnp.bfloat16),
        (((1,), (0,)), ((), ())),
        preferred_element_type=jnp.float32)
    a = sz_ref[...] + se_ref[...]
    d = a - 2.0 * m
    lane = jax.lax.broadcasted_iota(jnp.int32, d.shape, 1)

    mt = d.shape[0]
    acc_v = jnp.full((mt, 1), jnp.inf, jnp.float32)
    acc_i = jnp.zeros((mt, 1), jnp.int32)
    chosen_v = jnp.zeros((mt, 1), jnp.float32)
    for w0, w1 in ((0, 2736), (2736, 5472), (5472, _K)):
        mask = (lane >= w0) & (lane < w1)
        dw = jnp.where(mask, d, jnp.inf)
        mv = jnp.min(dw, axis=1, keepdims=True)
        iw = jnp.min(jnp.where(dw == mv, lane, jnp.int32(_K)), axis=1,
                     keepdims=True)
        keep = (acc_v < mv) | ((acc_v == mv) & (acc_i < iw))
        acc_i = jnp.where(keep, acc_i, iw)
        chosen_v = jnp.where(keep, chosen_v, mv)
        acc_v = jnp.where(keep, acc_v,
                          mv.astype(jnp.bfloat16).astype(jnp.float32))
    idx_ref[...] = acc_i

    @pl.when(pl.program_id(0) == 0)
    def _init():
        dsum_ref[...] = jnp.zeros_like(dsum_ref)

    dsum_ref[...] += jnp.sum(chosen_v)[None, None]


def _dist_argmin(z_flat, et, sz, se):
    return pl.pallas_call(
        _dist_argmin_body,
        grid=(_N // _MT,),
        in_specs=[
            pl.BlockSpec((_MT, _D), lambda i: (i, 0)),
            pl.BlockSpec((_D, _K), lambda i: (0, 0)),
            pl.BlockSpec((_MT, 1), lambda i: (i, 0)),
            pl.BlockSpec((1, _K), lambda i: (0, 0)),
        ],
        out_specs=[
            pl.BlockSpec((_MT, 1), lambda i: (i, 0)),
            pl.BlockSpec((1, 1), lambda i: (0, 0)),
        ],
        out_shape=[
            jax.ShapeDtypeStruct((_N, 1), jnp.int32),
            jax.ShapeDtypeStruct((1, 1), jnp.float32),
        ],
    )(z_flat, et, sz, se)


def _sc_gather_count(idx_flat, embedding):
    mesh = plsc.VectorSubcoreMesh(core_axis_name="c", subcore_axis_name="s")
    n_workers = 32
    bpw = _N // n_workers
    chunk = 128

    @functools.partial(
        pl.kernel,
        mesh=mesh,
        out_type=(
            jax.ShapeDtypeStruct((_N, _D), jnp.float32),
            jax.ShapeDtypeStruct((2, _K), jnp.float32),
        ),
        scratch_types=[
            pltpu.VMEM((bpw,), jnp.int32),
            pltpu.VMEM((chunk, _D), jnp.float32),
            pltpu.VMEM((bpw,), jnp.float32),
            pltpu.VMEM((_K,), jnp.float32),
            pltpu.VMEM_SHARED((_K,), jnp.float32),
            pltpu.SemaphoreType.DMA,
        ],
    )
    def kb(idx_hbm, table_hbm, zq_hbm, cnt_hbm,
           idxv, rows, onesv, stage, csh, sem):
        cid = jax.lax.axis_index("c")
        sid = jax.lax.axis_index("s")
        wid = sid * 2 + cid
        base = wid * bpw
        pltpu.sync_copy(idx_hbm.at[pl.ds(base, bpw)], idxv)
        for c in range(bpw // chunk):
            pltpu.async_copy(
                table_hbm.at[idxv.at[pl.ds(c * chunk, chunk)]], rows, sem
            ).wait()
            pltpu.sync_copy(rows, zq_hbm.at[pl.ds(base + c * chunk, chunk)])

        @pl.loop(0, bpw, step=16)
        def _(i):
            onesv[pl.ds(i, 16)] = jnp.ones((16,), jnp.float32)

        @pl.when(sid == 0)
        def _():
            @pl.loop(0, _K, step=16)
            def _(i):
                stage[pl.ds(i, 16)] = jnp.zeros((16,), jnp.float32)
            pltpu.sync_copy(stage, csh)

        plsc.subcore_barrier()
        pltpu.sync_copy(onesv, csh.at[idxv], add=True)
        plsc.subcore_barrier()

        @pl.when(sid == 0)
        def _():
            pltpu.sync_copy(csh, stage)
            pltpu.sync_copy(stage, cnt_hbm.at[cid])

    return kb(idx_flat, embedding)


def _assemble_body(in_ref, zq_ref, cnt_ref, out_ref, ent_ref):
    zin = in_ref[0]
    zq_t = jnp.transpose(zq_ref[0], (1, 0))
    out_ref[0] = zin + (zq_t - zin)

    @pl.when(pl.program_id(0) == 0)
    def _entropy():
        c = cnt_ref[0:1, :] + cnt_ref[1:2, :]
        em = c * (1.0 / float(_N))
        ent_ref[...] = jnp.sum(em * jnp.log(em + 1e-10))[None, None]


def _assemble(inputs_r, zq3, counts):
    b = inputs_r.shape[0]
    hw = inputs_r.shape[2]
    return pl.pallas_call(
        _assemble_body,
        grid=(b,),
        in_specs=[
            pl.BlockSpec((1, _D, hw), lambda i: (i, 0, 0)),
            pl.BlockSpec((1, hw, _D), lambda i: (i, 0, 0)),
            pl.BlockSpec((2, _K), lambda i: (0, 0)),
        ],
        out_specs=[
            pl.BlockSpec((1, _D, hw), lambda i: (i, 0, 0)),
            pl.BlockSpec((1, 1), lambda i: (0, 0)),
        ],
        out_shape=[
            jax.ShapeDtypeStruct((b, _D, hw), jnp.float32),
            jax.ShapeDtypeStruct((1, 1), jnp.float32),
        ],
    )(inputs_r, zq3, counts)


def kernel(inputs, embedding):
    b, c, h, w = inputs.shape
    z = jnp.transpose(inputs, (0, 2, 3, 1))
    z_flat = z.reshape(-1, c)
    sz = jnp.sum(z_flat ** 2, axis=1)
    se = jnp.sum(embedding ** 2, axis=1)
    et = embedding.T

    idxc, dsum = _dist_argmin(z_flat, et, sz.reshape(-1, 1), se.reshape(1, -1))
    idx = idxc.reshape(-1)

    zq_flat, counts = _sc_gather_count(idx, embedding)

    inputs_r = inputs.reshape(b, c, h * w)
    zq3 = zq_flat.reshape(b, h * w, c)
    outr, ent = _assemble(inputs_r, zq3, counts)

    z_q_out = outr.reshape(b, c, h, w)
    embedding_loss = (1.25 / float(_N * _D)) * dsum[0, 0]
    perplexity = jnp.exp(-ent[0, 0])
    return (embedding_loss, z_q_out, perplexity)


# speedup vs baseline: 1.3605x; 1.3605x over previous
import functools

import jax
import jax.numpy as jnp
from jax.experimental import pallas as pl
from jax.experimental.pallas import tpu as pltpu
from jax.experimental.pallas import tpu_sc as plsc

_K = 8192
_D = 256
_N = 16384
_MT = 128


def _dist_argmin_body(z_ref, et_ref, sz_ref, se_ref, idx_ref, dsum_ref):
    m = jax.lax.dot_general(
        z_ref[...].astype(jnp.bfloat16), et_ref[...].astype(jnp.bfloat16),
        (((1,), (0,)), ((), ())),
        preferred_element_type=jnp.float32)
    a = sz_ref[...] + se_ref[...]
    d = a - 2.0 * m
    lane = jax.lax.broadcasted_iota(jnp.int32, d.shape, 1)

    mt = d.shape[0]

    def window_minarg(chunks):
        run_v = run_c = None
        for c, lo, hi in chunks:
            dc = jax.lax.slice_in_dim(d, c * 128, (c + 1) * 128, axis=1)
            if lo > 0 or hi < 128:
                li = jax.lax.broadcasted_iota(jnp.int32, dc.shape, 1)
                dc = jnp.where((li >= lo) & (li < hi), dc,
                               jnp.float32(jnp.inf))
            if run_v is None:
                run_v = dc
                run_c = jnp.full(dc.shape, c, jnp.int32)
            else:
                cmp = dc < run_v
                run_c = jnp.where(cmp, jnp.int32(c), run_c)
                run_v = jnp.minimum(run_v, dc)
        li = jax.lax.broadcasted_iota(jnp.int32, run_v.shape, 1)
        gi = run_c * 128 + li
        mv = jnp.min(run_v, axis=1, keepdims=True)
        iw = jnp.min(jnp.where(run_v == mv, gi, jnp.int32(_K)), axis=1,
                     keepdims=True)
        return mv, iw

    full = lambda a, b: [(c, 0, 128) for c in range(a, b)]
    windows = [
        window_minarg(full(0, 21) + [(21, 0, 48)]),
        window_minarg([(21, 48, 128)] + full(22, 42) + [(42, 0, 96)]),
        window_minarg([(42, 96, 128)] + full(43, 64)),
    ]
    acc_v = jnp.full((mt, 1), jnp.inf, jnp.float32)
    acc_i = jnp.zeros((mt, 1), jnp.int32)
    chosen_v = jnp.zeros((mt, 1), jnp.float32)
    for mv, iw in windows:
        keep = (acc_v < mv) | ((acc_v == mv) & (acc_i < iw))
        acc_i = jnp.where(keep, acc_i, iw)
        chosen_v = jnp.where(keep, chosen_v, mv)
        acc_v = jnp.where(keep, acc_v,
                          mv.astype(jnp.bfloat16).astype(jnp.float32))
    idx_ref[...] = acc_i

    @pl.when(pl.program_id(0) == 0)
    def _init():
        dsum_ref[...] = jnp.zeros_like(dsum_ref)

    dsum_ref[...] += jnp.sum(chosen_v)[None, None]


def _dist_argmin(z_flat, et, sz, se):
    return pl.pallas_call(
        _dist_argmin_body,
        grid=(_N // _MT,),
        in_specs=[
            pl.BlockSpec((_MT, _D), lambda i: (i, 0)),
            pl.BlockSpec((_D, _K), lambda i: (0, 0)),
            pl.BlockSpec((_MT, 1), lambda i: (i, 0)),
            pl.BlockSpec((1, _K), lambda i: (0, 0)),
        ],
        out_specs=[
            pl.BlockSpec((_MT, 1), lambda i: (i, 0)),
            pl.BlockSpec((1, 1), lambda i: (0, 0)),
        ],
        out_shape=[
            jax.ShapeDtypeStruct((_N, 1), jnp.int32),
            jax.ShapeDtypeStruct((1, 1), jnp.float32),
        ],
    )(z_flat, et, sz, se)


def _sc_gather_count(idx_flat, embedding):
    mesh = plsc.VectorSubcoreMesh(core_axis_name="c", subcore_axis_name="s")
    n_workers = 32
    bpw = _N // n_workers
    chunk = 128

    @functools.partial(
        pl.kernel,
        mesh=mesh,
        out_type=(
            jax.ShapeDtypeStruct((_N, _D), jnp.float32),
            jax.ShapeDtypeStruct((2, _K), jnp.float32),
        ),
        scratch_types=[
            pltpu.VMEM((bpw,), jnp.int32),
            pltpu.VMEM((chunk, _D), jnp.float32),
            pltpu.VMEM((bpw,), jnp.float32),
            pltpu.VMEM((_K,), jnp.float32),
            pltpu.VMEM_SHARED((_K,), jnp.float32),
            pltpu.SemaphoreType.DMA,
        ],
    )
    def kb(idx_hbm, table_hbm, zq_hbm, cnt_hbm,
           idxv, rows, onesv, stage, csh, sem):
        cid = jax.lax.axis_index("c")
        sid = jax.lax.axis_index("s")
        wid = sid * 2 + cid
        base = wid * bpw
        pltpu.sync_copy(idx_hbm.at[pl.ds(base, bpw)], idxv)
        for c in range(bpw // chunk):
            pltpu.async_copy(
                table_hbm.at[idxv.at[pl.ds(c * chunk, chunk)]], rows, sem
            ).wait()
            pltpu.sync_copy(rows, zq_hbm.at[pl.ds(base + c * chunk, chunk)])

        @pl.loop(0, bpw, step=16)
        def _(i):
            onesv[pl.ds(i, 16)] = jnp.ones((16,), jnp.float32)

        @pl.when(sid == 0)
        def _():
            @pl.loop(0, _K, step=16)
            def _(i):
                stage[pl.ds(i, 16)] = jnp.zeros((16,), jnp.float32)
            pltpu.sync_copy(stage, csh)

        plsc.subcore_barrier()
        pltpu.sync_copy(onesv, csh.at[idxv], add=True)
        plsc.subcore_barrier()

        @pl.when(sid == 0)
        def _():
            pltpu.sync_copy(csh, stage)
            pltpu.sync_copy(stage, cnt_hbm.at[cid])

    return kb(idx_flat, embedding)


def _assemble_body(in_ref, zq_ref, cnt_ref, out_ref, ent_ref):
    zin = in_ref[0]
    zq_t = jnp.transpose(zq_ref[0], (1, 0))
    out_ref[0] = zin + (zq_t - zin)

    @pl.when(pl.program_id(0) == 0)
    def _entropy():
        c = cnt_ref[0:1, :] + cnt_ref[1:2, :]
        em = c * (1.0 / float(_N))
        ent_ref[...] = jnp.sum(em * jnp.log(em + 1e-10))[None, None]


def _assemble(inputs_r, zq3, counts):
    b = inputs_r.shape[0]
    hw = inputs_r.shape[2]
    return pl.pallas_call(
        _assemble_body,
        grid=(b,),
        in_specs=[
            pl.BlockSpec((1, _D, hw), lambda i: (i, 0, 0)),
            pl.BlockSpec((1, hw, _D), lambda i: (i, 0, 0)),
            pl.BlockSpec((2, _K), lambda i: (0, 0)),
        ],
        out_specs=[
            pl.BlockSpec((1, _D, hw), lambda i: (i, 0, 0)),
            pl.BlockSpec((1, 1), lambda i: (0, 0)),
        ],
        out_shape=[
            jax.ShapeDtypeStruct((b, _D, hw), jnp.float32),
            jax.ShapeDtypeStruct((1, 1), jnp.float32),
        ],
    )(inputs_r, zq3, counts)


def kernel(inputs, embedding):
    b, c, h, w = inputs.shape
    z = jnp.transpose(inputs, (0, 2, 3, 1))
    z_flat = z.reshape(-1, c)
    sz = jnp.sum(z_flat ** 2, axis=1)
    se = jnp.sum(embedding ** 2, axis=1)
    et = embedding.T

    idxc, dsum = _dist_argmin(z_flat, et, sz.reshape(-1, 1), se.reshape(1, -1))
    idx = idxc.reshape(-1)

    zq_flat, counts = _sc_gather_count(idx, embedding)

    inputs_r = inputs.reshape(b, c, h * w)
    zq3 = zq_flat.reshape(b, h * w, c)
    outr, ent = _assemble(inputs_r, zq3, counts)

    z_q_out = outr.reshape(b, c, h, w)
    embedding_loss = (1.25 / float(_N * _D)) * dsum[0, 0]
    perplexity = jnp.exp(-ent[0, 0])
    return (embedding_loss, z_q_out, perplexity)


# speedup vs baseline: 1.5333x; 1.1270x over previous
import functools

import jax
import jax.numpy as jnp
from jax.experimental import pallas as pl
from jax.experimental.pallas import tpu as pltpu
from jax.experimental.pallas import tpu_sc as plsc

_K = 8192
_D = 256
_N = 16384
_MT = 256


def _dist_argmin_body(z_ref, et_ref, sz_ref, se_ref, idx_ref, dsum_ref):
    m = jax.lax.dot_general(
        z_ref[...].astype(jnp.bfloat16), et_ref[...].astype(jnp.bfloat16),
        (((1,), (0,)), ((), ())),
        preferred_element_type=jnp.float32)
    a = sz_ref[...] + se_ref[...]
    d = a - 2.0 * m
    lane = jax.lax.broadcasted_iota(jnp.int32, d.shape, 1)

    mt = d.shape[0]

    def window_minarg(chunks):
        run_v = run_c = None
        for c, lo, hi in chunks:
            dc = jax.lax.slice_in_dim(d, c * 128, (c + 1) * 128, axis=1)
            if lo > 0 or hi < 128:
                li = jax.lax.broadcasted_iota(jnp.int32, dc.shape, 1)
                dc = jnp.where((li >= lo) & (li < hi), dc,
                               jnp.float32(jnp.inf))
            if run_v is None:
                run_v = dc
                run_c = jnp.full(dc.shape, c, jnp.int32)
            else:
                cmp = dc < run_v
                run_c = jnp.where(cmp, jnp.int32(c), run_c)
                run_v = jnp.minimum(run_v, dc)
        li = jax.lax.broadcasted_iota(jnp.int32, run_v.shape, 1)
        gi = run_c * 128 + li
        mv = jnp.min(run_v, axis=1, keepdims=True)
        iw = jnp.min(jnp.where(run_v == mv, gi, jnp.int32(_K)), axis=1,
                     keepdims=True)
        return mv, iw

    full = lambda a, b: [(c, 0, 128) for c in range(a, b)]
    windows = [
        window_minarg(full(0, 21) + [(21, 0, 48)]),
        window_minarg([(21, 48, 128)] + full(22, 42) + [(42, 0, 96)]),
        window_minarg([(42, 96, 128)] + full(43, 64)),
    ]
    acc_v = jnp.full((mt, 1), jnp.inf, jnp.float32)
    acc_i = jnp.zeros((mt, 1), jnp.int32)
    chosen_v = jnp.zeros((mt, 1), jnp.float32)
    for mv, iw in windows:
        keep = (acc_v < mv) | ((acc_v == mv) & (acc_i < iw))
        acc_i = jnp.where(keep, acc_i, iw)
        chosen_v = jnp.where(keep, chosen_v, mv)
        acc_v = jnp.where(keep, acc_v,
                          mv.astype(jnp.bfloat16).astype(jnp.float32))
    idx_ref[...] = acc_i

    @pl.when(pl.program_id(0) == 0)
    def _init():
        dsum_ref[...] = jnp.zeros_like(dsum_ref)

    dsum_ref[...] += jnp.sum(chosen_v)[None, None]


def _dist_argmin(z_flat, et, sz, se):
    return pl.pallas_call(
        _dist_argmin_body,
        grid=(_N // _MT,),
        in_specs=[
            pl.BlockSpec((_MT, _D), lambda i: (i, 0)),
            pl.BlockSpec((_D, _K), lambda i: (0, 0)),
            pl.BlockSpec((_MT, 1), lambda i: (i, 0)),
            pl.BlockSpec((1, _K), lambda i: (0, 0)),
        ],
        out_specs=[
            pl.BlockSpec((_MT, 1), lambda i: (i, 0)),
            pl.BlockSpec((1, 1), lambda i: (0, 0)),
        ],
        out_shape=[
            jax.ShapeDtypeStruct((_N, 1), jnp.int32),
            jax.ShapeDtypeStruct((1, 1), jnp.float32),
        ],
    )(z_flat, et, sz, se)


def _sc_gather_count(idx_flat, embedding):
    mesh = plsc.VectorSubcoreMesh(core_axis_name="c", subcore_axis_name="s")
    n_workers = 32
    bpw = _N // n_workers
    chunk = 128

    @functools.partial(
        pl.kernel,
        mesh=mesh,
        out_type=(
            jax.ShapeDtypeStruct((_N, _D), jnp.float32),
            jax.ShapeDtypeStruct((2, _K), jnp.float32),
        ),
        scratch_types=[
            pltpu.VMEM((bpw,), jnp.int32),
            pltpu.VMEM((chunk, _D), jnp.float32),
            pltpu.VMEM((bpw,), jnp.float32),
            pltpu.VMEM((_K,), jnp.float32),
            pltpu.VMEM_SHARED((_K,), jnp.float32),
            pltpu.SemaphoreType.DMA,
        ],
    )
    def kb(idx_hbm, table_hbm, zq_hbm, cnt_hbm,
           idxv, rows, onesv, stage, csh, sem):
        cid = jax.lax.axis_index("c")
        sid = jax.lax.axis_index("s")
        wid = sid * 2 + cid
        base = wid * bpw
        pltpu.sync_copy(idx_hbm.at[pl.ds(base, bpw)], idxv)
        for c in range(bpw // chunk):
            pltpu.async_copy(
                table_hbm.at[idxv.at[pl.ds(c * chunk, chunk)]], rows, sem
            ).wait()
            pltpu.sync_copy(rows, zq_hbm.at[pl.ds(base + c * chunk, chunk)])

        @pl.loop(0, bpw, step=16)
        def _(i):
            onesv[pl.ds(i, 16)] = jnp.ones((16,), jnp.float32)

        @pl.when(sid == 0)
        def _():
            @pl.loop(0, _K, step=16)
            def _(i):
                stage[pl.ds(i, 16)] = jnp.zeros((16,), jnp.float32)
            pltpu.sync_copy(stage, csh)

        plsc.subcore_barrier()
        pltpu.sync_copy(onesv, csh.at[idxv], add=True)
        plsc.subcore_barrier()

        @pl.when(sid == 0)
        def _():
            pltpu.sync_copy(csh, stage)
            pltpu.sync_copy(stage, cnt_hbm.at[cid])

    return kb(idx_flat, embedding)


def _assemble_body(in_ref, zq_ref, cnt_ref, out_ref, ent_ref):
    zin = in_ref[0]
    zq_t = jnp.transpose(zq_ref[0], (1, 0))
    out_ref[0] = zin + (zq_t - zin)

    @pl.when(pl.program_id(0) == 0)
    def _entropy():
        c = cnt_ref[0:1, :] + cnt_ref[1:2, :]
        em = c * (1.0 / float(_N))
        ent_ref[...] = jnp.sum(em * jnp.log(em + 1e-10))[None, None]


def _assemble(inputs_r, zq3, counts):
    b = inputs_r.shape[0]
    hw = inputs_r.shape[2]
    return pl.pallas_call(
        _assemble_body,
        grid=(b,),
        in_specs=[
            pl.BlockSpec((1, _D, hw), lambda i: (i, 0, 0)),
            pl.BlockSpec((1, hw, _D), lambda i: (i, 0, 0)),
            pl.BlockSpec((2, _K), lambda i: (0, 0)),
        ],
        out_specs=[
            pl.BlockSpec((1, _D, hw), lambda i: (i, 0, 0)),
            pl.BlockSpec((1, 1), lambda i: (0, 0)),
        ],
        out_shape=[
            jax.ShapeDtypeStruct((b, _D, hw), jnp.float32),
            jax.ShapeDtypeStruct((1, 1), jnp.float32),
        ],
    )(inputs_r, zq3, counts)


def kernel(inputs, embedding):
    b, c, h, w = inputs.shape
    z = jnp.transpose(inputs, (0, 2, 3, 1))
    z_flat = z.reshape(-1, c)
    sz = jnp.sum(z_flat ** 2, axis=1)
    se = jnp.sum(embedding ** 2, axis=1)
    et = embedding.T

    idxc, dsum = _dist_argmin(z_flat, et, sz.reshape(-1, 1), se.reshape(1, -1))
    idx = idxc.reshape(-1)

    zq_flat, counts = _sc_gather_count(idx, embedding)

    inputs_r = inputs.reshape(b, c, h * w)
    zq3 = zq_flat.reshape(b, h * w, c)
    outr, ent = _assemble(inputs_r, zq3, counts)

    z_q_out = outr.reshape(b, c, h, w)
    embedding_loss = (1.25 / float(_N * _D)) * dsum[0, 0]
    perplexity = jnp.exp(-ent[0, 0])
    return (embedding_loss, z_q_out, perplexity)
